# baseline (speedup 1.0000x reference)
import functools

import jax
import jax.numpy as jnp
from jax.experimental import pallas as pl
from jax.experimental.pallas import tpu as pltpu

D_BLK = 512
TOK_BLK = 1024


def _fold_kernel(mk_ref, w_ref, mt_ref):
    mt_ref[...] = jax.lax.dot_general(
        w_ref[...].astype(jnp.bfloat16), mk_ref[...].astype(jnp.bfloat16),
        dimension_numbers=(((0,), (1,)), ((), ())),
        preferred_element_type=jnp.float32,
    )


def _router_kernel(temp_ref, emb_ref, mt_ref, mk_ref, b_ref, out_ref):
    logits = jax.lax.dot_general(
        emb_ref[...].astype(jnp.bfloat16), mt_ref[...].astype(jnp.bfloat16),
        dimension_numbers=(((1,), (0,)), ((), ())),
        preferred_element_type=jnp.float32,
    )
    bias = jax.lax.dot_general(
        b_ref[...], mk_ref[...],
        dimension_numbers=(((1,), (1,)), ((), ())),
        preferred_element_type=jnp.float32,
        precision=jax.lax.Precision.HIGHEST,
    )
    temperature = jnp.maximum(jnp.exp(temp_ref[0]), 1e-4)
    inv_scale = 1.0 / (64.0 * temperature)
    logits = (logits + bias) * inv_scale
    m = jnp.max(logits, axis=-1, keepdims=True)
    e = jnp.exp(logits - m)
    out_ref[...] = e / jnp.sum(e, axis=-1, keepdims=True)


@jax.jit
def kernel(embedding, W, b, module_keys, log_temperature):
    n_tokens, d_model = embedding.shape
    n_modules = module_keys.shape[0]

    mt = pl.pallas_call(
        _fold_kernel,
        grid=(d_model // D_BLK,),
        in_specs=[
            pl.BlockSpec((n_modules, d_model), lambda j: (0, 0)),
            pl.BlockSpec((d_model, D_BLK), lambda j: (0, j)),
        ],
        out_specs=pl.BlockSpec((D_BLK, n_modules), lambda j: (j, 0)),
        out_shape=jax.ShapeDtypeStruct((d_model, n_modules), jnp.float32),
        compiler_params=pltpu.CompilerParams(
            dimension_semantics=("parallel",)),
    )(module_keys, W)

    temp = jnp.reshape(log_temperature, (1,)).astype(jnp.float32)
    b2 = jnp.reshape(b, (1, d_model))
    out = pl.pallas_call(
        _router_kernel,
        grid=(n_tokens // TOK_BLK,),
        in_specs=[
            pl.BlockSpec(memory_space=pltpu.SMEM),
            pl.BlockSpec((TOK_BLK, d_model), lambda i: (i, 0)),
            pl.BlockSpec((d_model, n_modules), lambda i: (0, 0)),
            pl.BlockSpec((n_modules, d_model), lambda i: (0, 0)),
            pl.BlockSpec((1, d_model), lambda i: (0, 0)),
        ],
        out_specs=pl.BlockSpec((TOK_BLK, n_modules), lambda i: (i, 0)),
        out_shape=jax.ShapeDtypeStruct((n_tokens, n_modules), jnp.float32),
        compiler_params=pltpu.CompilerParams(
            dimension_semantics=("parallel",)),
    )(temp, embedding, mt, module_keys, b2)
    return out


# speedup vs baseline: 1.0034x; 1.0034x over previous
import jax
import jax.numpy as jnp
from jax.experimental import pallas as pl
from jax.experimental.pallas import tpu as pltpu

D_BLK = 512
TOK_BLK = 1024
N_FOLD = 4096 // D_BLK


def _fused_kernel(temp_ref, mk_ref, w_ref, emb_ref, b_ref, out_ref, mt_ref):
    t = pl.program_id(0)

    @pl.when(t < N_FOLD)
    def _fold():
        stripe = jax.lax.dot_general(
            w_ref[...].astype(jnp.bfloat16), mk_ref[...].astype(jnp.bfloat16),
            dimension_numbers=(((0,), (1,)), ((), ())),
            preferred_element_type=jnp.float32,
        )
        mt_ref[pl.ds(t * D_BLK, D_BLK), :] = stripe.astype(jnp.bfloat16)

    @pl.when(t >= N_FOLD)
    def _route():
        logits = jax.lax.dot_general(
            emb_ref[...].astype(jnp.bfloat16), mt_ref[...],
            dimension_numbers=(((1,), (0,)), ((), ())),
            preferred_element_type=jnp.float32,
        )
        bias = jax.lax.dot_general(
            b_ref[...], mk_ref[...],
            dimension_numbers=(((1,), (1,)), ((), ())),
            preferred_element_type=jnp.float32,
        )
        temperature = jnp.maximum(jnp.exp(temp_ref[0]), 1e-4)
        inv_scale = 1.0 / (64.0 * temperature)
        scaled = (logits + bias) * inv_scale
        m = jnp.max(scaled, axis=-1, keepdims=True)
        e = jnp.exp(scaled - m)
        out_ref[...] = e / jnp.sum(e, axis=-1, keepdims=True)


@jax.jit
def kernel(embedding, W, b, module_keys, log_temperature):
    n_tokens, d_model = embedding.shape
    n_modules = module_keys.shape[0]
    n_tok_blocks = n_tokens // TOK_BLK

    temp = jnp.reshape(log_temperature, (1,)).astype(jnp.float32)
    b2 = jnp.reshape(b, (1, d_model))
    return pl.pallas_call(
        _fused_kernel,
        grid=(N_FOLD + n_tok_blocks,),
        in_specs=[
            pl.BlockSpec(memory_space=pltpu.SMEM),
            pl.BlockSpec((n_modules, d_model), lambda t: (0, 0)),
            pl.BlockSpec((d_model, D_BLK),
                         lambda t: (0, jnp.minimum(t, N_FOLD - 1))),
            pl.BlockSpec((TOK_BLK, d_model),
                         lambda t: (jnp.maximum(t - N_FOLD, 0), 0)),
            pl.BlockSpec((1, d_model), lambda t: (0, 0)),
        ],
        out_specs=pl.BlockSpec((TOK_BLK, n_modules),
                               lambda t: (jnp.maximum(t - N_FOLD, 0), 0)),
        out_shape=jax.ShapeDtypeStruct((n_tokens, n_modules), jnp.float32),
        scratch_shapes=[pltpu.VMEM((d_model, n_modules), jnp.bfloat16)],
        compiler_params=pltpu.CompilerParams(
            dimension_semantics=("arbitrary",)),
    )(temp, module_keys, W, embedding, b2)
